# Initial kernel scaffold; baseline (speedup 1.0000x reference)
import jax
import jax.numpy as jnp
from jax.experimental import pallas as pl


def kernel(inputs, user_table, item_table):
    raise NotImplementedError("write your pallas kernel here")


# speedup vs baseline: 1.1276x; 1.1276x over previous
import functools

import jax
import jax.numpy as jnp
from jax import lax
from jax.experimental import pallas as pl
from jax.experimental.pallas import tpu as pltpu
from jax.experimental.pallas import tpu_sc as plsc

_NC = 2
_NS = 16
_NW = _NC * _NS
_LANES = 16

_EMBED = 128
_BATCH = 16384
_B_PER_W = _BATCH // _NW
_CHUNK = 128
_NCHUNKS = _B_PER_W // _CHUNK


def _sc_body(uidx_hbm, iidx_hbm, utab_hbm, itab_hbm, out_hbm,
             uidx_v, iidx_v, u_v, i_v, acc_v, out_v, sem_u, sem_i):
    wid = lax.axis_index("s") * _NC + lax.axis_index("c")

    pltpu.sync_copy(uidx_hbm.at[wid], uidx_v)
    pltpu.sync_copy(iidx_hbm.at[wid], iidx_v)

    lanes = lax.iota(jnp.int32, _LANES)

    for c in range(_NCHUNKS):
        cp_u = pltpu.async_copy(utab_hbm.at[uidx_v.at[c]], u_v, sem_u)
        cp_i = pltpu.async_copy(itab_hbm.at[iidx_v.at[c]], i_v, sem_i)
        cp_u.wait()
        cp_i.wait()

        def group_body(g, _, c=c):
            base = g * _LANES
            for rr in range(_LANES):
                r = base + rr
                acc = u_v[r, pl.ds(0, _LANES)] * i_v[r, pl.ds(0, _LANES)]
                for j in range(1, _EMBED // _LANES):
                    acc = acc + (u_v[r, pl.ds(j * _LANES, _LANES)]
                                 * i_v[r, pl.ds(j * _LANES, _LANES)])
                acc_v[rr] = acc
            res = plsc.load_gather(acc_v, [lanes, jnp.zeros((_LANES,), jnp.int32)])
            for j in range(1, _LANES):
                res = res + plsc.load_gather(
                    acc_v, [lanes, jnp.full((_LANES,), j, jnp.int32)])
            out_v[c, pl.ds(base, _LANES)] = res
            return 0

        lax.fori_loop(0, _CHUNK // _LANES, group_body, 0)

    pltpu.sync_copy(out_v, out_hbm.at[wid])


@jax.jit
def _cf_dot(uidx, iidx, user_table, item_table):
    mesh = plsc.VectorSubcoreMesh(core_axis_name="c", subcore_axis_name="s",
                                  num_cores=_NC, num_subcores=_NS)
    k = pl.kernel(
        _sc_body,
        out_type=jax.ShapeDtypeStruct((_NW, _NCHUNKS, _CHUNK), jnp.float32),
        mesh=mesh,
        scratch_types=[
            pltpu.VMEM((_NCHUNKS, _CHUNK), jnp.int32),
            pltpu.VMEM((_NCHUNKS, _CHUNK), jnp.int32),
            pltpu.VMEM((_CHUNK, _EMBED), jnp.float32),
            pltpu.VMEM((_CHUNK, _EMBED), jnp.float32),
            pltpu.VMEM((_LANES, _LANES), jnp.float32),
            pltpu.VMEM((_NCHUNKS, _CHUNK), jnp.float32),
            pltpu.SemaphoreType.DMA,
            pltpu.SemaphoreType.DMA,
        ],
        compiler_params=pltpu.CompilerParams(needs_layout_passes=False),
    )
    return k(uidx, iidx, user_table, item_table)


def kernel(inputs, user_table, item_table):
    idx = inputs.astype(jnp.int32)
    uidx = idx[:, 0].reshape(_NW, _NCHUNKS, _CHUNK)
    iidx = idx[:, 1].reshape(_NW, _NCHUNKS, _CHUNK)
    out = _cf_dot(uidx, iidx, user_table, item_table)
    return out.reshape(_BATCH)


# speedup vs baseline: 1.2635x; 1.1205x over previous
import functools

import jax
import jax.numpy as jnp
from jax import lax
from jax.experimental import pallas as pl
from jax.experimental.pallas import tpu as pltpu
from jax.experimental.pallas import tpu_sc as plsc

_NC = 2
_NS = 16
_NW = _NC * _NS
_LANES = 16

_EMBED = 128
_BATCH = 16384
_B_PER_W = _BATCH // _NW
_CHUNK = 128
_NCHUNKS = _B_PER_W // _CHUNK


def _sc_body(uidx_hbm, iidx_hbm, utab_hbm, itab_hbm, out_hbm,
             uidx_v, iidx_v, u0_v, u1_v, i0_v, i1_v, acc_v, out_v,
             sem_u0, sem_u1, sem_i0, sem_i1):
    wid = lax.axis_index("s") * _NC + lax.axis_index("c")

    pltpu.sync_copy(uidx_hbm.at[wid], uidx_v)
    pltpu.sync_copy(iidx_hbm.at[wid], iidx_v)

    lanes = lax.iota(jnp.int32, _LANES)
    u_bufs = (u0_v, u1_v)
    i_bufs = (i0_v, i1_v)
    u_sems = (sem_u0, sem_u1)
    i_sems = (sem_i0, sem_i1)

    def issue(c):
        s = c % 2
        return (pltpu.async_copy(utab_hbm.at[uidx_v.at[c]], u_bufs[s], u_sems[s]),
                pltpu.async_copy(itab_hbm.at[iidx_v.at[c]], i_bufs[s], i_sems[s]))

    pending = issue(0)
    for c in range(_NCHUNKS):
        cp_u, cp_i = pending
        if c + 1 < _NCHUNKS:
            nxt = issue(c + 1)
        cp_u.wait()
        cp_i.wait()
        if c + 1 < _NCHUNKS:
            pending = nxt
        u_v = u_bufs[c % 2]
        i_v = i_bufs[c % 2]

        def group_body(g, _, c=c, u_v=u_v, i_v=i_v):
            base = g * _LANES
            for rr in range(_LANES):
                r = base + rr
                acc = u_v[r, pl.ds(0, _LANES)] * i_v[r, pl.ds(0, _LANES)]
                for j in range(1, _EMBED // _LANES):
                    acc = acc + (u_v[r, pl.ds(j * _LANES, _LANES)]
                                 * i_v[r, pl.ds(j * _LANES, _LANES)])
                acc_v[rr] = acc
            res = plsc.load_gather(acc_v, [lanes, jnp.zeros((_LANES,), jnp.int32)])
            for j in range(1, _LANES):
                res = res + plsc.load_gather(
                    acc_v, [lanes, jnp.full((_LANES,), j, jnp.int32)])
            out_v[c, pl.ds(base, _LANES)] = res
            return 0

        lax.fori_loop(0, _CHUNK // _LANES, group_body, 0)

    pltpu.sync_copy(out_v, out_hbm.at[wid])


@jax.jit
def _cf_dot(uidx, iidx, user_table, item_table):
    mesh = plsc.VectorSubcoreMesh(core_axis_name="c", subcore_axis_name="s",
                                  num_cores=_NC, num_subcores=_NS)
    k = pl.kernel(
        _sc_body,
        out_type=jax.ShapeDtypeStruct((_NW, _NCHUNKS, _CHUNK), jnp.float32),
        mesh=mesh,
        scratch_types=[
            pltpu.VMEM((_NCHUNKS, _CHUNK), jnp.int32),
            pltpu.VMEM((_NCHUNKS, _CHUNK), jnp.int32),
            pltpu.VMEM((_CHUNK, _EMBED), jnp.float32),
            pltpu.VMEM((_CHUNK, _EMBED), jnp.float32),
            pltpu.VMEM((_CHUNK, _EMBED), jnp.float32),
            pltpu.VMEM((_CHUNK, _EMBED), jnp.float32),
            pltpu.VMEM((_LANES, _LANES), jnp.float32),
            pltpu.VMEM((_NCHUNKS, _CHUNK), jnp.float32),
            pltpu.SemaphoreType.DMA,
            pltpu.SemaphoreType.DMA,
            pltpu.SemaphoreType.DMA,
            pltpu.SemaphoreType.DMA,
        ],
        compiler_params=pltpu.CompilerParams(needs_layout_passes=False),
    )
    return k(uidx, iidx, user_table, item_table)


def kernel(inputs, user_table, item_table):
    idx = inputs.astype(jnp.int32)
    uidx = idx[:, 0].reshape(_NW, _NCHUNKS, _CHUNK)
    iidx = idx[:, 1].reshape(_NW, _NCHUNKS, _CHUNK)
    out = _cf_dot(uidx, iidx, user_table, item_table)
    return out.reshape(_BATCH)
